# baseline (device time: 80215 ns/iter reference)
import jax
import jax.numpy as jnp
from jax import lax
from jax.experimental import pallas as pl
from jax.experimental.pallas import tpu as pltpu

NY, NZ = 4, 4
NREP = NY * NZ
M = 2048
D = 2048
TM = 512
TN = 512
TGRID = D // TN
HM = TM // 2

RB = 352
XB = TM - RB
HB = RB // 2

HAM = [
    (0, 0), (0, 1), (0, 2), (0, 3),
    (1, 3), (1, 2), (1, 1),
    (2, 1), (2, 2), (2, 3),
    (3, 3), (3, 2), (3, 1), (3, 0),
    (2, 0), (1, 0),
]
assert len(HAM) == NREP and len(set(HAM)) == NREP
for _a, _b in zip(HAM, HAM[1:] + HAM[:1]):
    assert abs(_a[0] - _b[0]) + abs(_a[1] - _b[1]) == 1, (_a, _b)

N_FULL = 7
N_LONG = 8


def _ring_pos(y, z):
    p = jnp.int32(0)
    for i, (yy, zz) in enumerate(HAM):
        p = jnp.where((y == yy) & (z == zz), jnp.int32(i), p)
    return p


def _coords_at(p, offset):
    ny = jnp.int32(0)
    nz = jnp.int32(0)
    for i in range(NREP):
        yy, zz = HAM[(i + offset) % NREP]
        ny = jnp.where(p == i, jnp.int32(yy), ny)
        nz = jnp.where(p == i, jnp.int32(zz), nz)
    return ny, nz


def _gemm_tile(tile_idx, dy, w):
    _, k = dy.shape
    bk = 2048
    kt = k // bk

    def body(idx_ref, dy_ref, w_ref, out_ref):
        ki = pl.program_id(0)

        @pl.when(ki == 0)
        def _():
            out_ref[...] = jnp.zeros_like(out_ref)

        a = dy_ref[...].astype(jnp.bfloat16)
        b = w_ref[...].astype(jnp.bfloat16)
        out_ref[...] += lax.dot_general(
            a, b, (((1,), (1,)), ((), ())),
            preferred_element_type=jnp.float32,
        )

    grid_spec = pltpu.PrefetchScalarGridSpec(
        num_scalar_prefetch=1,
        grid=(kt,),
        in_specs=[
            pl.BlockSpec((TM, bk), lambda ki, idx: (idx[0], ki)),
            pl.BlockSpec((TN, bk), lambda ki, idx: (idx[1], ki)),
        ],
        out_specs=pl.BlockSpec((TM, TN), lambda ki, idx: (0, 0)),
    )
    return pl.pallas_call(
        body,
        grid_spec=grid_spec,
        out_shape=jax.ShapeDtypeStruct((TM, TN), jnp.float32),
    )(tile_idx, dy, w)


def _x_reduce_3link_allgather(partial):

    def body(p_ref, out_ref, xsend, xrecv, rt, cw, ccw,
             stg_c, stg_cc, shr_c, shr_cc, ant_stg, ant_recv,
             sx, rx, cw_s, cw_r, ccw_s, ccw_r,
             stgs_c, shrr_c, stgs_cc, shrr_cc, ant_s, ant_r,
             cr_c, cr_cc):
        my_x = lax.axis_index("x")
        my_y = lax.axis_index("y")
        my_z = lax.axis_index("z")
        p = _ring_pos(my_y, my_z)
        ny1, nz1 = _coords_at(p, 1)
        py1, pz1 = _coords_at(p, -1)
        nxt = (my_x, ny1, nz1)
        prv = (my_x, py1, pz1)
        xpartner = (1 - my_x, my_y, my_z)

        rb = my_x * XB
        xbase = (1 - my_x) * RB
        voff = my_x * (HB - XB)

        barrier = pltpu.get_barrier_semaphore()
        for dev in (nxt, prv, xpartner):
            pl.semaphore_signal(
                barrier, inc=1, device_id=dev,
                device_id_type=pl.DeviceIdType.MESH,
            )
        pl.semaphore_wait(barrier, 3)

        orow_p = (p // TGRID) * TM
        ocol_p = (p % TGRID) * TN

        def tile_rows(o):
            return (o // TGRID) * TM, (o % TGRID) * TN

        xsend[0] = p_ref[0:HM, :].astype(jnp.bfloat16)
        xsend[1] = p_ref[HM:TM, :].astype(jnp.bfloat16)
        xr = [
            pltpu.make_async_remote_copy(
                src_ref=xsend.at[v], dst_ref=xrecv.at[v],
                send_sem=sx.at[v], recv_sem=rx.at[v],
                device_id=xpartner, device_id_type=pl.DeviceIdType.MESH,
            )
            for v in range(2)
        ]
        xr[0].start()
        xr[1].start()
        xr[0].wait()
        red0 = p_ref[0:HM, :] + xrecv[0].astype(jnp.float32)
        rt[0:HM] = red0.astype(jnp.bfloat16)
        out_ref[pl.ds(orow_p, HM), pl.ds(ocol_p, TN)] = red0
        xr[1].wait()
        red1 = p_ref[HM:TM, :] + xrecv[1].astype(jnp.float32)
        rt[HM:TM] = red1.astype(jnp.bfloat16)
        out_ref[pl.ds(orow_p + HM, HM), pl.ds(ocol_p, TN)] = red1

        cw[0, 0] = rt[pl.ds(rb, HB), :]
        cw[1, 0] = rt[pl.ds(rb + HB, HB), :]
        ccw[0, 0] = cw[0, 0]
        ccw[1, 0] = cw[1, 0]

        pipes = [
            ("cwA", cw, 0, nxt, N_LONG, cw_s, cw_r, -1),
            ("cwB", cw, 1, nxt, N_FULL, cw_s, cw_r, -1),
            ("ccwA", ccw, 0, prv, N_FULL, ccw_s, ccw_r, +1),
            ("ccwB", ccw, 1, prv, N_LONG, ccw_s, ccw_r, +1),
        ]

        def ring_desc(cfg, j):
            _, buf, v, dev, _, ss, rs, _ = cfg
            return pltpu.make_async_remote_copy(
                src_ref=buf.at[v, j % 2], dst_ref=buf.at[v, (j + 1) % 2],
                send_sem=ss.at[v, j % 2], recv_sem=rs.at[v, (j + 1) % 2],
                device_id=dev, device_id_type=pl.DeviceIdType.MESH,
            )

        def share_desc(stg, shr, ss, rs, j):
            return pltpu.make_async_remote_copy(
                src_ref=stg.at[j % 2], dst_ref=shr.at[j % 2],
                send_sem=ss.at[j % 2], recv_sem=rs.at[j % 2],
                device_id=xpartner, device_id_type=pl.DeviceIdType.MESH,
            )

        shares = [
            ("c", stg_c, shr_c, stgs_c, shrr_c, cr_c, cw, -1),
            ("cc", stg_cc, shr_cc, stgs_cc, shrr_cc, cr_cc, ccw, +1),
        ]

        for cfg in pipes:
            ring_desc(cfg, 0).start()

        N_SH = N_FULL

        for h in range(1, N_LONG + 1):
            j = h - 1
            for cfg in pipes:
                if j < cfg[4]:
                    ring_desc(cfg, j).wait()
            for cfg in pipes:
                if h < cfg[4]:
                    ring_desc(cfg, h).start()
            if j < N_SH:
                for _, stg, shr, ss, rs, cr, buf, _sgn in shares:
                    if j >= 2:
                        share_desc(stg, shr, ss, rs, j - 2).wait_send()
                        pl.semaphore_wait(cr, 1)
                    stg[j % 2] = buf[my_x, h % 2, pl.ds(voff, XB), :]
                    share_desc(stg, shr, ss, rs, j).start()
            for cfg in pipes:
                name, buf, v, _, n, _, _, sgn = cfg
                if j < n:
                    o = (p + sgn * (j + 1)) % NREP
                    orow, ocol = tile_rows(o)
                    out_ref[pl.ds(orow + rb + v * HB, HB),
                            pl.ds(ocol, TN)] = buf[v, h % 2].astype(jnp.float32)
            if j < N_SH:
                for _, stg, shr, ss, rs, cr, _buf, sgn in shares:
                    share_desc(stg, shr, ss, rs, j).wait_recv()
                    o = (p + sgn * (j + 1)) % NREP
                    orow, ocol = tile_rows(o)
                    out_ref[pl.ds(orow + xbase, XB),
                            pl.ds(ocol, TN)] = shr[j % 2].astype(jnp.float32)
                    if j + 2 < N_SH:
                        pl.semaphore_signal(
                            cr, inc=1, device_id=xpartner,
                            device_id_type=pl.DeviceIdType.MESH,
                        )

        @pl.when(my_x == 0)
        def _():
            ant_stg[...] = cw[0, N_LONG % 2, pl.ds(0, XB), :]

        @pl.when(my_x == 1)
        def _():
            ant_stg[...] = ccw[1, N_LONG % 2, pl.ds(HB - XB, XB), :]

        ant = pltpu.make_async_remote_copy(
            src_ref=ant_stg, dst_ref=ant_recv,
            send_sem=ant_s, recv_sem=ant_r,
            device_id=xpartner, device_id_type=pl.DeviceIdType.MESH,
        )
        ant.start()
        ant.wait_send()
        ant.wait_recv()
        o = (p + N_LONG) % NREP
        orow, ocol = tile_rows(o)
        out_ref[pl.ds(orow + xbase, XB),
                pl.ds(ocol, TN)] = ant_recv[...].astype(jnp.float32)

        for _, stg, shr, ss, rs, _cr, _buf, _sgn in shares:
            share_desc(stg, shr, ss, rs, N_SH - 2).wait_send()
            share_desc(stg, shr, ss, rs, N_SH - 1).wait_send()

    return pl.pallas_call(
        body,
        out_shape=jax.ShapeDtypeStruct((M, D), jnp.float32),
        in_specs=[pl.BlockSpec(memory_space=pltpu.VMEM)],
        out_specs=pl.BlockSpec(memory_space=pltpu.VMEM),
        scratch_shapes=[
            pltpu.VMEM((2, HM, TN), jnp.bfloat16),
            pltpu.VMEM((2, HM, TN), jnp.bfloat16),
            pltpu.VMEM((TM, TN), jnp.bfloat16),
            pltpu.VMEM((2, 2, HB, TN), jnp.bfloat16),
            pltpu.VMEM((2, 2, HB, TN), jnp.bfloat16),
            pltpu.VMEM((2, XB, TN), jnp.bfloat16),
            pltpu.VMEM((2, XB, TN), jnp.bfloat16),
            pltpu.VMEM((2, XB, TN), jnp.bfloat16),
            pltpu.VMEM((2, XB, TN), jnp.bfloat16),
            pltpu.VMEM((XB, TN), jnp.bfloat16),
            pltpu.VMEM((XB, TN), jnp.bfloat16),
            pltpu.SemaphoreType.DMA((2,)),
            pltpu.SemaphoreType.DMA((2,)),
            pltpu.SemaphoreType.DMA((2, 2)),
            pltpu.SemaphoreType.DMA((2, 2)),
            pltpu.SemaphoreType.DMA((2, 2)),
            pltpu.SemaphoreType.DMA((2, 2)),
            pltpu.SemaphoreType.DMA((2,)),
            pltpu.SemaphoreType.DMA((2,)),
            pltpu.SemaphoreType.DMA((2,)),
            pltpu.SemaphoreType.DMA((2,)),
            pltpu.SemaphoreType.DMA,
            pltpu.SemaphoreType.DMA,
            pltpu.SemaphoreType.REGULAR,
            pltpu.SemaphoreType.REGULAR,
        ],
        compiler_params=pltpu.CompilerParams(collective_id=0),
    )(partial)


def kernel(dy, W):
    my_y = lax.axis_index("y")
    my_z = lax.axis_index("z")
    p = _ring_pos(my_y, my_z)
    tile_idx = jnp.stack([p // TGRID, p % TGRID]).astype(jnp.int32)
    partial = _gemm_tile(tile_idx, dy, W)
    return _x_reduce_3link_allgather(partial)


# device time: 80156 ns/iter; 1.0007x vs baseline; 1.0007x over previous
import jax
import jax.numpy as jnp
from jax import lax
from jax.experimental import pallas as pl
from jax.experimental.pallas import tpu as pltpu

NY, NZ = 4, 4
NREP = NY * NZ
M = 2048
D = 2048
TM = 512
TN = 512
TGRID = D // TN
HM = TM // 2

RB = 352
XB = TM - RB
HB = RB // 2

HAM = [
    (0, 0), (0, 1), (0, 2), (0, 3),
    (1, 3), (1, 2), (1, 1),
    (2, 1), (2, 2), (2, 3),
    (3, 3), (3, 2), (3, 1), (3, 0),
    (2, 0), (1, 0),
]
assert len(HAM) == NREP and len(set(HAM)) == NREP
for _a, _b in zip(HAM, HAM[1:] + HAM[:1]):
    assert abs(_a[0] - _b[0]) + abs(_a[1] - _b[1]) == 1, (_a, _b)

N_FULL = 7
N_LONG = 8


def _ring_pos(y, z):
    p = jnp.int32(0)
    for i, (yy, zz) in enumerate(HAM):
        p = jnp.where((y == yy) & (z == zz), jnp.int32(i), p)
    return p


def _coords_at(p, offset):
    ny = jnp.int32(0)
    nz = jnp.int32(0)
    for i in range(NREP):
        yy, zz = HAM[(i + offset) % NREP]
        ny = jnp.where(p == i, jnp.int32(yy), ny)
        nz = jnp.where(p == i, jnp.int32(zz), nz)
    return ny, nz


def _gemm_tile(tile_idx, dy, w):
    _, k = dy.shape
    bk = 2048
    kt = k // bk

    def body(idx_ref, dy_ref, w_ref, out_ref):
        ki = pl.program_id(0)

        @pl.when(ki == 0)
        def _():
            out_ref[...] = jnp.zeros_like(out_ref)

        a = dy_ref[...].astype(jnp.bfloat16)
        b = w_ref[...].astype(jnp.bfloat16)
        out_ref[...] += lax.dot_general(
            a, b, (((1,), (1,)), ((), ())),
            preferred_element_type=jnp.float32,
        )

    grid_spec = pltpu.PrefetchScalarGridSpec(
        num_scalar_prefetch=1,
        grid=(kt,),
        in_specs=[
            pl.BlockSpec((TM, bk), lambda ki, idx: (idx[0], ki)),
            pl.BlockSpec((TN, bk), lambda ki, idx: (idx[1], ki)),
        ],
        out_specs=pl.BlockSpec((TM, TN), lambda ki, idx: (0, 0)),
    )
    return pl.pallas_call(
        body,
        grid_spec=grid_spec,
        out_shape=jax.ShapeDtypeStruct((TM, TN), jnp.float32),
    )(tile_idx, dy, w)


def _x_reduce_3link_allgather(partial):

    def body(p_ref, out_ref, xsend, xrecv, rt, cw, ccw,
             stg_c, stg_cc, shr_c, shr_cc, ant_stg, ant_recv,
             sx, rx, cw_s, cw_r, ccw_s, ccw_r,
             stgs_c, shrr_c, stgs_cc, shrr_cc, ant_s, ant_r):
        my_x = lax.axis_index("x")
        my_y = lax.axis_index("y")
        my_z = lax.axis_index("z")
        p = _ring_pos(my_y, my_z)
        ny1, nz1 = _coords_at(p, 1)
        py1, pz1 = _coords_at(p, -1)
        nxt = (my_x, ny1, nz1)
        prv = (my_x, py1, pz1)
        xpartner = (1 - my_x, my_y, my_z)

        rb = my_x * XB
        xbase = (1 - my_x) * RB
        voff = my_x * (HB - XB)

        barrier = pltpu.get_barrier_semaphore()
        for dev in (nxt, prv, xpartner):
            pl.semaphore_signal(
                barrier, inc=1, device_id=dev,
                device_id_type=pl.DeviceIdType.MESH,
            )
        pl.semaphore_wait(barrier, 3)

        orow_p = (p // TGRID) * TM
        ocol_p = (p % TGRID) * TN

        def tile_rows(o):
            return (o // TGRID) * TM, (o % TGRID) * TN

        xsend[0] = p_ref[0:HM, :].astype(jnp.bfloat16)
        xsend[1] = p_ref[HM:TM, :].astype(jnp.bfloat16)
        xr = [
            pltpu.make_async_remote_copy(
                src_ref=xsend.at[v], dst_ref=xrecv.at[v],
                send_sem=sx.at[v], recv_sem=rx.at[v],
                device_id=xpartner, device_id_type=pl.DeviceIdType.MESH,
            )
            for v in range(2)
        ]
        xr[0].start()
        xr[1].start()
        xr[0].wait()
        red0 = p_ref[0:HM, :] + xrecv[0].astype(jnp.float32)
        rt[0:HM] = red0.astype(jnp.bfloat16)
        out_ref[pl.ds(orow_p, HM), pl.ds(ocol_p, TN)] = red0
        xr[1].wait()
        red1 = p_ref[HM:TM, :] + xrecv[1].astype(jnp.float32)
        rt[HM:TM] = red1.astype(jnp.bfloat16)
        out_ref[pl.ds(orow_p + HM, HM), pl.ds(ocol_p, TN)] = red1

        cw[0, 0] = rt[pl.ds(rb, HB), :]
        cw[1, 0] = rt[pl.ds(rb + HB, HB), :]
        ccw[0, 0] = cw[0, 0]
        ccw[1, 0] = cw[1, 0]

        pipes = [
            ("cwA", cw, 0, nxt, N_LONG, cw_s, cw_r, -1),
            ("cwB", cw, 1, nxt, N_FULL, cw_s, cw_r, -1),
            ("ccwA", ccw, 0, prv, N_FULL, ccw_s, ccw_r, +1),
            ("ccwB", ccw, 1, prv, N_LONG, ccw_s, ccw_r, +1),
        ]

        def ring_desc(cfg, j):
            _, buf, v, dev, _, ss, rs, _ = cfg
            return pltpu.make_async_remote_copy(
                src_ref=buf.at[v, j % 2], dst_ref=buf.at[v, (j + 1) % 2],
                send_sem=ss.at[v, j % 2], recv_sem=rs.at[v, (j + 1) % 2],
                device_id=dev, device_id_type=pl.DeviceIdType.MESH,
            )

        def share_desc(stg, shr, ss, rs, j):
            return pltpu.make_async_remote_copy(
                src_ref=stg.at[j], dst_ref=shr.at[j],
                send_sem=ss.at[j], recv_sem=rs.at[j],
                device_id=xpartner, device_id_type=pl.DeviceIdType.MESH,
            )

        shares = [
            ("c", stg_c, shr_c, stgs_c, shrr_c, cw, -1),
            ("cc", stg_cc, shr_cc, stgs_cc, shrr_cc, ccw, +1),
        ]

        for cfg in pipes:
            ring_desc(cfg, 0).start()

        N_SH = N_FULL

        for h in range(1, N_LONG + 1):
            j = h - 1
            for cfg in pipes:
                if j < cfg[4]:
                    ring_desc(cfg, j).wait()
            for cfg in pipes:
                if h < cfg[4]:
                    ring_desc(cfg, h).start()
            if j < N_SH:
                for _, stg, shr, ss, rs, buf, _sgn in shares:
                    stg[j] = buf[my_x, h % 2, pl.ds(voff, XB), :]
                    share_desc(stg, shr, ss, rs, j).start()
            for cfg in pipes:
                name, buf, v, _, n, _, _, sgn = cfg
                if j < n:
                    o = (p + sgn * (j + 1)) % NREP
                    orow, ocol = tile_rows(o)
                    out_ref[pl.ds(orow + rb + v * HB, HB),
                            pl.ds(ocol, TN)] = buf[v, h % 2].astype(jnp.float32)
            jl = j - 2
            if 0 <= jl < N_SH:
                for _, stg, shr, ss, rs, _buf, sgn in shares:
                    share_desc(stg, shr, ss, rs, jl).wait_recv()
                    o = (p + sgn * (jl + 1)) % NREP
                    orow, ocol = tile_rows(o)
                    out_ref[pl.ds(orow + xbase, XB),
                            pl.ds(ocol, TN)] = shr[jl].astype(jnp.float32)

        for _, stg, shr, ss, rs, _buf, sgn in shares:
            share_desc(stg, shr, ss, rs, N_SH - 1).wait_recv()
            o = (p + sgn * N_SH) % NREP
            orow, ocol = tile_rows(o)
            out_ref[pl.ds(orow + xbase, XB),
                    pl.ds(ocol, TN)] = shr[N_SH - 1].astype(jnp.float32)

        @pl.when(my_x == 0)
        def _():
            ant_stg[...] = cw[0, N_LONG % 2, pl.ds(0, XB), :]

        @pl.when(my_x == 1)
        def _():
            ant_stg[...] = ccw[1, N_LONG % 2, pl.ds(HB - XB, XB), :]

        ant = pltpu.make_async_remote_copy(
            src_ref=ant_stg, dst_ref=ant_recv,
            send_sem=ant_s, recv_sem=ant_r,
            device_id=xpartner, device_id_type=pl.DeviceIdType.MESH,
        )
        ant.start()
        ant.wait_send()
        ant.wait_recv()
        o = (p + N_LONG) % NREP
        orow, ocol = tile_rows(o)
        out_ref[pl.ds(orow + xbase, XB),
                pl.ds(ocol, TN)] = ant_recv[...].astype(jnp.float32)

        for _, stg, shr, ss, rs, _buf, _sgn in shares:
            for j in range(N_SH):
                share_desc(stg, shr, ss, rs, j).wait_send()

    return pl.pallas_call(
        body,
        out_shape=jax.ShapeDtypeStruct((M, D), jnp.float32),
        in_specs=[pl.BlockSpec(memory_space=pltpu.VMEM)],
        out_specs=pl.BlockSpec(memory_space=pltpu.VMEM),
        scratch_shapes=[
            pltpu.VMEM((2, HM, TN), jnp.bfloat16),
            pltpu.VMEM((2, HM, TN), jnp.bfloat16),
            pltpu.VMEM((TM, TN), jnp.bfloat16),
            pltpu.VMEM((2, 2, HB, TN), jnp.bfloat16),
            pltpu.VMEM((2, 2, HB, TN), jnp.bfloat16),
            pltpu.VMEM((N_FULL, XB, TN), jnp.bfloat16),
            pltpu.VMEM((N_FULL, XB, TN), jnp.bfloat16),
            pltpu.VMEM((N_FULL, XB, TN), jnp.bfloat16),
            pltpu.VMEM((N_FULL, XB, TN), jnp.bfloat16),
            pltpu.VMEM((XB, TN), jnp.bfloat16),
            pltpu.VMEM((XB, TN), jnp.bfloat16),
            pltpu.SemaphoreType.DMA((2,)),
            pltpu.SemaphoreType.DMA((2,)),
            pltpu.SemaphoreType.DMA((2, 2)),
            pltpu.SemaphoreType.DMA((2, 2)),
            pltpu.SemaphoreType.DMA((2, 2)),
            pltpu.SemaphoreType.DMA((2, 2)),
            pltpu.SemaphoreType.DMA((N_FULL,)),
            pltpu.SemaphoreType.DMA((N_FULL,)),
            pltpu.SemaphoreType.DMA((N_FULL,)),
            pltpu.SemaphoreType.DMA((N_FULL,)),
            pltpu.SemaphoreType.DMA,
            pltpu.SemaphoreType.DMA,
        ],
        compiler_params=pltpu.CompilerParams(collective_id=0),
    )(partial)


def kernel(dy, W):
    my_y = lax.axis_index("y")
    my_z = lax.axis_index("z")
    p = _ring_pos(my_y, my_z)
    tile_idx = jnp.stack([p // TGRID, p % TGRID]).astype(jnp.int32)
    partial = _gemm_tile(tile_idx, dy, W)
    return _x_reduce_3link_allgather(partial)


# device time: 72293 ns/iter; 1.1096x vs baseline; 1.1088x over previous
import jax
import jax.numpy as jnp
from jax import lax
from jax.experimental import pallas as pl
from jax.experimental.pallas import tpu as pltpu

NY, NZ = 4, 4
NREP = NY * NZ
M = 2048
D = 2048
TM = 512
TN = 512
TGRID = D // TN

HAM = [
    (0, 0), (0, 1), (0, 2), (0, 3),
    (1, 3), (1, 2), (1, 1),
    (2, 1), (2, 2), (2, 3),
    (3, 3), (3, 2), (3, 1), (3, 0),
    (2, 0), (1, 0),
]
assert len(HAM) == NREP and len(set(HAM)) == NREP
for _a, _b in zip(HAM, HAM[1:] + HAM[:1]):
    assert abs(_a[0] - _b[0]) + abs(_a[1] - _b[1]) == 1, (_a, _b)

N_CW = NREP // 2
N_CCW = NREP - 1 - N_CW


def _ring_pos(y, z):
    p = jnp.int32(0)
    for i, (yy, zz) in enumerate(HAM):
        p = jnp.where((y == yy) & (z == zz), jnp.int32(i), p)
    return p


def _coords_at(p, offset):
    ny = jnp.int32(0)
    nz = jnp.int32(0)
    for i in range(NREP):
        yy, zz = HAM[(i + offset) % NREP]
        ny = jnp.where(p == i, jnp.int32(yy), ny)
        nz = jnp.where(p == i, jnp.int32(zz), nz)
    return ny, nz


def _gemm_tile(tile_idx, dy, w):
    _, k = dy.shape
    bk = 1024
    kt = k // bk

    def body(idx_ref, dy_ref, w_ref, out_ref):
        ki = pl.program_id(0)

        @pl.when(ki == 0)
        def _():
            out_ref[...] = jnp.zeros_like(out_ref)

        a = dy_ref[...].astype(jnp.bfloat16)
        b = w_ref[...].astype(jnp.bfloat16)
        out_ref[...] += lax.dot_general(
            a, b, (((1,), (1,)), ((), ())),
            preferred_element_type=jnp.float32,
        )

    grid_spec = pltpu.PrefetchScalarGridSpec(
        num_scalar_prefetch=1,
        grid=(kt,),
        in_specs=[
            pl.BlockSpec((TM, bk), lambda ki, idx: (idx[0], ki)),
            pl.BlockSpec((TN, bk), lambda ki, idx: (idx[1], ki)),
        ],
        out_specs=pl.BlockSpec((TM, TN), lambda ki, idx: (0, 0)),
    )
    return pl.pallas_call(
        body,
        grid_spec=grid_spec,
        out_shape=jax.ShapeDtypeStruct((TM, TN), jnp.float32),
    )(tile_idx, dy, w)


HM = TM // 2


def _x_reduce_yz_allgather(partial):

    def _store_half(out_ref, o, v, block16):
        orow = (o // TGRID) * TM + v * HM
        ocol = (o % TGRID) * TN
        out_ref[pl.ds(orow, HM), pl.ds(ocol, TN)] = block16.astype(jnp.bfloat16)

    def body(p_ref, out_ref, xsend, xrecv, cw, ccw,
             sx, rx, cw_s, cw_r, ccw_s, ccw_r):
        my_x = lax.axis_index("x")
        my_y = lax.axis_index("y")
        my_z = lax.axis_index("z")
        p = _ring_pos(my_y, my_z)
        ny1, nz1 = _coords_at(p, 1)
        py1, pz1 = _coords_at(p, -1)
        nxt = (my_x, ny1, nz1)
        prv = (my_x, py1, pz1)
        xpartner = (1 - my_x, my_y, my_z)

        barrier = pltpu.get_barrier_semaphore()
        for dev in (nxt, prv, xpartner):
            pl.semaphore_signal(
                barrier, inc=1, device_id=dev,
                device_id_type=pl.DeviceIdType.MESH,
            )
        pl.semaphore_wait(barrier, 3)

        def ring_rdma(bufs, sems_s, sems_r, dev, v, h):
            s, r = h % 2, (h + 1) % 2
            return pltpu.make_async_remote_copy(
                src_ref=bufs.at[v, s], dst_ref=bufs.at[v, r],
                send_sem=sems_s.at[v, s], recv_sem=sems_r.at[v, r],
                device_id=dev, device_id_type=pl.DeviceIdType.MESH,
            )

        xsend[0] = p_ref[0:HM, :].astype(jnp.bfloat16)
        xsend[1] = p_ref[HM:TM, :].astype(jnp.bfloat16)
        xr = [
            pltpu.make_async_remote_copy(
                src_ref=xsend.at[v], dst_ref=xrecv.at[v],
                send_sem=sx.at[v], recv_sem=rx.at[v],
                device_id=xpartner, device_id_type=pl.DeviceIdType.MESH,
            )
            for v in range(2)
        ]
        xr[0].start()
        xr[1].start()

        pipes = [
            (cw, 0, cw_s, cw_r, nxt, 8, -1),
            (cw, 1, cw_s, cw_r, nxt, 7, -1),
            (ccw, 0, ccw_s, ccw_r, prv, 7, +1),
            (ccw, 1, ccw_s, ccw_r, prv, 8, +1),
        ]

        for v in range(2):
            xr[v].wait()
            red = p_ref[pl.ds(v * HM, HM), :] + xrecv[v].astype(jnp.float32)
            red16 = red.astype(jnp.bfloat16)
            cw[v, 0] = red16
            ccw[v, 0] = red16
            for buf, pv, ss, rs, dev, _n, _sgn in pipes:
                if pv == v:
                    ring_rdma(buf, ss, rs, dev, pv, 0).start()
            _store_half(out_ref, p, v, red)

        for h in range(1, 9):
            j = h - 1
            for v in range(2):
                vp = [cfg for cfg in pipes if cfg[1] == v]
                for buf, _v, ss, rs, dev, n, sgn in vp:
                    if j < n:
                        ring_rdma(buf, ss, rs, dev, v, j).wait()
                for buf, _v, ss, rs, dev, n, sgn in vp:
                    if h < n:
                        ring_rdma(buf, ss, rs, dev, v, h).start()
                for buf, _v, ss, rs, dev, n, sgn in vp:
                    if j < n:
                        _store_half(out_ref, (p + sgn * (j + 1)) % NREP, v,
                                    buf[v, h % 2])

    return pl.pallas_call(
        body,
        out_shape=jax.ShapeDtypeStruct((M, D), jnp.bfloat16),
        in_specs=[pl.BlockSpec(memory_space=pltpu.VMEM)],
        out_specs=pl.BlockSpec(memory_space=pltpu.VMEM),
        scratch_shapes=[
            pltpu.VMEM((2, HM, TN), jnp.bfloat16),
            pltpu.VMEM((2, HM, TN), jnp.bfloat16),
            pltpu.VMEM((2, 2, HM, TN), jnp.bfloat16),
            pltpu.VMEM((2, 2, HM, TN), jnp.bfloat16),
            pltpu.SemaphoreType.DMA((2,)),
            pltpu.SemaphoreType.DMA((2,)),
            pltpu.SemaphoreType.DMA((2, 2)),
            pltpu.SemaphoreType.DMA((2, 2)),
            pltpu.SemaphoreType.DMA((2, 2)),
            pltpu.SemaphoreType.DMA((2, 2)),
        ],
        compiler_params=pltpu.CompilerParams(collective_id=0),
    )(partial)


def kernel(dy, W):
    my_y = lax.axis_index("y")
    my_z = lax.axis_index("z")
    p = _ring_pos(my_y, my_z)
    tile_idx = jnp.stack([p // TGRID, p % TGRID]).astype(jnp.int32)
    partial = _gemm_tile(tile_idx, dy, W)
    return _x_reduce_yz_allgather(partial)
